# baseline (speedup 1.0000x reference)
import jax
import jax.numpy as jnp
from jax.experimental import pallas as pl
from jax.experimental.pallas import tpu as pltpu

B = 4096
D = 16
MARGIN = 2.0
QUANT = 30
ROWS = 2048
NT = B // ROWS
CHUNK = 64
BITS = 16


def _body(o1e_ref, o2_ref, o2te_ref, o2td_ref, rsq_ref, out_ref, vsel_ref):
    i = pl.program_id(0)

    @pl.when(i == 0)
    def _init():
        out_ref[0, 0] = jnp.float32(0.0)

    o1e = o1e_ref[...]
    o2 = o2_ref[...]
    o2te = o2te_ref[...]
    o2td = o2td_ref[...]

    pos = jnp.sum((o2 - o1e[:, :D]) ** 2, axis=1)

    mm = MARGIN * MARGIN
    dn = (((1,), (0,)), ((), ()))

    dsm = jax.lax.dot_general(
        o1e, o2td, dn, preferred_element_type=jnp.float32)
    rloc = jax.lax.broadcasted_iota(jnp.int32, (ROWS, 1), 0)
    cloc = jax.lax.broadcasted_iota(jnp.int32, (ROWS, ROWS), 1)
    ddiag = jnp.max(jnp.where(cloc == rloc, dsm, -3e38), axis=1)

    d2 = jax.lax.dot_general(
        o1e, o2te, dn, preferred_element_type=jnp.float32)
    w = (jnp.where(d2 < ddiag[:, None], 1, 0)
         + jnp.where(d2 < mm, 8192, 0))
    ws = jnp.sum(w, axis=1)
    rdiag = jax.lax.bitwise_and(ws, 8191)
    c4 = jax.lax.shift_right_logical(ws, 13)

    rn = rsq_ref[:, 0]
    rn2 = rsq_ref[:, 1]
    rsel = jnp.where(rdiag == rn, rn2, rn)
    vsel_ref[0, :] = jnp.full((ROWS,), mm, jnp.float32)

    need = c4 > rsel
    for ch in range(ROWS // CHUNK):
        lo = ch * CHUNK

        @pl.when(jnp.any(need[lo:lo + CHUNK]))
        def _exact_select(lo=lo):
            bits = jax.lax.bitcast_convert_type(
                d2[lo:lo + CHUNK, :], jnp.int32)
            rs = rsel[lo:lo + CHUNK]

            def bstep(k, t):
                cand = t + (1 << (30 - k))
                c = jnp.sum((bits < cand[:, None]).astype(jnp.int32), axis=1)
                return jnp.where(c <= rs, cand, t)

            tfin = jax.lax.fori_loop(
                0, BITS, bstep, jnp.zeros((CHUNK,), jnp.int32))
            vsel_ref[0, pl.ds(lo, CHUNK)] = jax.lax.bitcast_convert_type(
                tfin, jnp.float32)

    vs = jnp.maximum(vsel_ref[0, :], 0.0)
    neg = jnp.maximum(MARGIN - jnp.sqrt(vs), 0.0)
    out_ref[0, 0] += jnp.sum(pos) + jnp.sum(neg)


@jax.jit
def _run(output1, output2):
    rn = jax.random.randint(jax.random.key(1), (B,), 0, QUANT)
    rn2 = (rn + 1) % QUANT
    rsq = jnp.stack([rn, rn2], axis=1).astype(jnp.int32)
    ones = jnp.ones((B, 1), jnp.float32)
    sq1 = jnp.sum(output1 * output1, axis=1, keepdims=True)
    sq2 = jnp.sum(output2 * output2, axis=1, keepdims=True)
    o1e = jnp.concatenate([output1, ones, sq1], axis=1)
    o2te = jnp.concatenate([-2.0 * output2, sq2, ones], axis=1).T
    o2 = output2
    total = pl.pallas_call(
        _body,
        grid=(NT,),
        in_specs=[
            pl.BlockSpec((ROWS, D + 2), lambda i: (i, 0)),
            pl.BlockSpec((ROWS, D), lambda i: (i, 0)),
            pl.BlockSpec((D + 2, B), lambda i: (0, 0)),
            pl.BlockSpec((D + 2, ROWS), lambda i: (0, i)),
            pl.BlockSpec((ROWS, 2), lambda i: (i, 0)),
        ],
        out_specs=pl.BlockSpec(
            (1, 1), lambda i: (0, 0), memory_space=pltpu.SMEM),
        out_shape=jax.ShapeDtypeStruct((1, 1), jnp.float32),
        scratch_shapes=[pltpu.VMEM((1, ROWS), jnp.float32)],
    )(o1e, o2, o2te, o2te, rsq)
    return total[0, 0] / jnp.float32(B)


def kernel(output1, output2):
    return _run(output1, output2)


# speedup vs baseline: 1.0289x; 1.0289x over previous
import jax
import jax.numpy as jnp
from jax.experimental import pallas as pl
from jax.experimental.pallas import tpu as pltpu

B = 4096
D = 16
MARGIN = 2.0
QUANT = 30
ROWS = 1024
NT = B // ROWS
CHUNK = 64
BITS = 16


def _body(o1e_ref, o2_ref, o2te_ref, o2td_ref, rsq_ref, out_ref, vsel_ref):
    i = pl.program_id(0)

    @pl.when(i == 0)
    def _init():
        out_ref[0, 0] = jnp.float32(0.0)

    o1e = o1e_ref[...]
    o2 = o2_ref[...]
    o2te = o2te_ref[...]
    o2td = o2td_ref[...]

    pos = jnp.sum((o2 - o1e[:, :D]) ** 2, axis=1)

    mm = MARGIN * MARGIN
    dn = (((1,), (0,)), ((), ()))

    dsm = jax.lax.dot_general(
        o1e, o2td, dn, preferred_element_type=jnp.float32)
    rloc = jax.lax.broadcasted_iota(jnp.int32, (ROWS, 1), 0)
    cloc = jax.lax.broadcasted_iota(jnp.int32, (ROWS, ROWS), 1)
    ddiag = jnp.max(jnp.where(cloc == rloc, dsm, -3e38), axis=1)

    d2 = jax.lax.dot_general(
        o1e, o2te, dn, preferred_element_type=jnp.float32)
    w = (jnp.where(d2 < ddiag[:, None], 1, 0)
         + jnp.where(d2 < mm, 8192, 0))
    ws = jnp.sum(w, axis=1)
    rdiag = jax.lax.bitwise_and(ws, 8191)
    c4 = jax.lax.shift_right_logical(ws, 13)

    rn = rsq_ref[:, 0]
    rn2 = rsq_ref[:, 1]
    rsel = jnp.where(rdiag == rn, rn2, rn)
    vsel_ref[0, :] = jnp.full((ROWS,), mm, jnp.float32)

    need = c4 > rsel
    for ch in range(ROWS // CHUNK):
        lo = ch * CHUNK

        @pl.when(jnp.any(need[lo:lo + CHUNK]))
        def _exact_select(lo=lo):
            bits = jax.lax.bitcast_convert_type(
                d2[lo:lo + CHUNK, :], jnp.int32)
            rs = rsel[lo:lo + CHUNK]

            def bstep(k, t):
                cand = t + (1 << (30 - k))
                c = jnp.sum((bits < cand[:, None]).astype(jnp.int32), axis=1)
                return jnp.where(c <= rs, cand, t)

            tfin = jax.lax.fori_loop(
                0, BITS, bstep, jnp.zeros((CHUNK,), jnp.int32))
            vsel_ref[0, pl.ds(lo, CHUNK)] = jax.lax.bitcast_convert_type(
                tfin, jnp.float32)

    vs = jnp.maximum(vsel_ref[0, :], 0.0)
    neg = jnp.maximum(MARGIN - jnp.sqrt(vs), 0.0)
    out_ref[0, 0] += jnp.sum(pos) + jnp.sum(neg)


@jax.jit
def _run(output1, output2):
    rn = jax.random.randint(jax.random.key(1), (B,), 0, QUANT)
    rn2 = (rn + 1) % QUANT
    rsq = jnp.stack([rn, rn2], axis=1).astype(jnp.int32)
    ones = jnp.ones((B, 1), jnp.float32)
    sq1 = jnp.sum(output1 * output1, axis=1, keepdims=True)
    sq2 = jnp.sum(output2 * output2, axis=1, keepdims=True)
    o1e = jnp.concatenate([output1, ones, sq1], axis=1)
    o2te = jnp.concatenate([-2.0 * output2, sq2, ones], axis=1).T
    o2 = output2
    total = pl.pallas_call(
        _body,
        grid=(NT,),
        in_specs=[
            pl.BlockSpec((ROWS, D + 2), lambda i: (i, 0)),
            pl.BlockSpec((ROWS, D), lambda i: (i, 0)),
            pl.BlockSpec((D + 2, B), lambda i: (0, 0)),
            pl.BlockSpec((D + 2, ROWS), lambda i: (0, i)),
            pl.BlockSpec((ROWS, 2), lambda i: (i, 0)),
        ],
        out_specs=pl.BlockSpec(
            (1, 1), lambda i: (0, 0), memory_space=pltpu.SMEM),
        out_shape=jax.ShapeDtypeStruct((1, 1), jnp.float32),
        scratch_shapes=[pltpu.VMEM((1, ROWS), jnp.float32)],
    )(o1e, o2, o2te, o2te, rsq)
    return total[0, 0] / jnp.float32(B)


def kernel(output1, output2):
    return _run(output1, output2)
